# baseline (speedup 1.0000x reference)
import functools

import jax
import jax.numpy as jnp
import numpy as np
from jax import lax
from jax.experimental import pallas as pl
from jax.experimental.pallas import tpu as pltpu
from jax.experimental.pallas import tpu_sc as plsc

_N = 16
_W = 384
_H = 384
_NUM_SAMPLES = 2000
_TOTAL = _N * _NUM_SAMPLES
_NS = 16
_NW = 32
_PAD = 32768
_HALF = 512
_NTASK = _PAD // _HALF
_LANES = 16
_HCHUNKS = _HALF // _LANES
_FAST_C = 1
_THRESH2 = float(np.float32(0.1) * np.float32(0.1))


def _prep_constants():
    key = jax.random.key(42)
    k1, k2, k3 = jax.random.split(key, 3)
    us, vs = [], []
    for k in (k1, k2, k3):
        ku, kv = jax.random.split(k)
        us.append(jax.random.randint(ku, (_N, _NUM_SAMPLES), 0, _W))
        vs.append(jax.random.randint(kv, (_N, _NUM_SAMPLES), 0, _H))
    boff = (jnp.arange(_N, dtype=jnp.int32) * (_W * _H))[:, None]
    idx = [(boff + u * _H + v).reshape(-1) for u, v in zip(us, vs)]
    uf = [(u.astype(jnp.float32) / _W).reshape(-1) for u in us]
    vf = [(v.astype(jnp.float32) / _H).reshape(-1) for v in vs]
    dx12 = uf[1] - uf[0]
    dy12 = vf[1] - vf[0]
    dx13 = uf[2] - uf[0]
    dy13 = vf[2] - vf[0]
    npad = _PAD - _TOTAL

    def pad(a):
        return jnp.concatenate([a, jnp.zeros((npad,), a.dtype)])

    return tuple(
        pad(a) for a in (idx[0], idx[1], idx[2], dx12, dy12, dx13, dy13))


def _pack_constants(vals):
    i1, i2, i3, ax, ay, bx, by = vals
    icat = jnp.stack([v.reshape(_NTASK, _HALF) for v in (i1, i2, i3)],
                     axis=1).reshape(-1)
    geo = jnp.stack([v.reshape(_NTASK, _HALF) for v in (ax, ay, bx, by)],
                    axis=1).reshape(-1)
    return icat, geo


@functools.cache
def _host_constants():
    try:
        try:
            dev = jax.devices("cpu")[0]
        except RuntimeError:
            dev = None
        with jax.ensure_compile_time_eval():
            if dev is not None:
                with jax.default_device(dev):
                    vals = _pack_constants(_prep_constants())
            else:
                vals = _pack_constants(_prep_constants())
        return tuple(np.asarray(v) for v in jax.device_get(vals))
    except Exception:
        return None


def _rsqrt(s):
    i = lax.bitcast_convert_type(s, jnp.int32)
    i = jnp.int32(0x5F3759DF) - (i >> 1)
    y = lax.bitcast_convert_type(i, jnp.float32)
    for _ in range(3):
        y = y * (1.5 - 0.5 * s * y * y)
    return y


@functools.cache
def _make_sc_kernel():
    mesh = plsc.VectorSubcoreMesh(core_axis_name="c", subcore_axis_name="s")
    f32, i32 = jnp.float32, jnp.int32

    @functools.partial(
        pl.kernel,
        mesh=mesh,
        out_type=jax.ShapeDtypeStruct((_NW, 2 * _LANES), f32),
        scratch_types=[
            pltpu.VMEM((_HALF,), i32),
            pltpu.VMEM((_HALF,), i32),
            pltpu.VMEM((_HALF,), i32),
            pltpu.VMEM((_HALF,), i32),
            pltpu.VMEM((_HALF,), i32),
            pltpu.VMEM((_HALF,), i32),
            pltpu.VMEM((4 * _HALF,), f32),
            pltpu.VMEM((4 * _HALF,), f32),
            pltpu.VMEM((_HALF,), f32),
            pltpu.VMEM((_HALF,), f32),
            pltpu.VMEM((_HALF,), f32),
            pltpu.VMEM((_HALF,), f32),
            pltpu.VMEM((_HALF,), f32),
            pltpu.VMEM((_HALF,), f32),
            pltpu.VMEM((_HALF,), f32),
            pltpu.VMEM((_HALF,), f32),
            pltpu.VMEM((_HALF,), f32),
            pltpu.VMEM((_HALF,), f32),
            pltpu.VMEM((_HALF,), f32),
            pltpu.VMEM((_HALF,), f32),
            pltpu.VMEM((2 * _LANES,), f32),
            pltpu.SemaphoreType.DMA,
            pltpu.SemaphoreType.DMA,
            pltpu.SemaphoreType.DMA,
            pltpu.SemaphoreType.DMA,
            pltpu.SemaphoreType.DMA,
            pltpu.SemaphoreType.DMA,
            pltpu.SemaphoreType.DMA,
            pltpu.SemaphoreType.DMA,
            pltpu.SemaphoreType.DMA,
        ],
    )
    def sc_kernel(pred_hbm, targ_hbm, icat_h, geo_h, out_h,
                  i1a, i2a, i3a, i1b, i2b, i3b, geo_a, geo_b,
                  gp1a, gp2a, gp3a, gt1a, gt2a, gt3a,
                  gp1b, gp2b, gp3b, gt1b, gt2b, gt3b,
                  acc, sem_i0, sem_i1, sem_i2,
                  sem_geo0, sem_geo1, sem_geo2,
                  sem_g0, sem_g1, sem_g2):
        c = lax.axis_index("c")
        s = lax.axis_index("s")
        is_fast = c == _FAST_C
        wid = c * _NS + s
        t0 = jnp.where(is_fast, _NS + 3 * s, s)
        t1 = t0 + 1
        t2 = t0 + 2

        iset = ((i1a, i2a, i3a), (i1b, i2b, i3b))
        gset = ((gp1a, gp2a, gp3a, gt1a, gt2a, gt3a),
                (gp1b, gp2b, gp3b, gt1b, gt2b, gt3b))
        geos = (geo_a, geo_b)
        isems = (sem_i0, sem_i1, sem_i2)
        gsems = (sem_geo0, sem_geo1, sem_geo2)
        dsems = (sem_g0, sem_g1, sem_g2)

        def idx_copies(t, k):
            ib = 3 * _HALF * t
            return [pltpu.make_async_copy(
                        icat_h.at[pl.ds(ib + j * _HALF, _HALF)],
                        iset[k % 2][j], isems[k])
                    for j in range(3)]

        def geo_copy(t, k):
            return pltpu.make_async_copy(
                geo_h.at[pl.ds(4 * _HALF * t, 4 * _HALF)], geos[k % 2],
                gsems[k])

        def gather_copies(k):
            ivs = iset[k % 2]
            gvs = gset[k % 2]
            cps = [pltpu.make_async_copy(pred_hbm.at[ivs[j]], gvs[j],
                                         dsems[k]) for j in range(3)]
            cps += [pltpu.make_async_copy(targ_hbm.at[ivs[j]], gvs[j + 3],
                                          dsems[k]) for j in range(3)]
            return cps

        def compute(k):
            gp1, gp2, gp3, gt1, gt2, gt3 = gset[k % 2]
            geo = geos[k % 2]

            def body(i, carry):
                s_acc, c_acc = carry
                off = i * _LANES
                d1p = gp1[pl.ds(off, _LANES)]
                d2p = gp2[pl.ds(off, _LANES)]
                d3p = gp3[pl.ds(off, _LANES)]
                t1v = gt1[pl.ds(off, _LANES)]
                t2v = gt2[pl.ds(off, _LANES)]
                t3v = gt3[pl.ds(off, _LANES)]
                ax = geo[pl.ds(off, _LANES)]
                ay = geo[pl.ds(_HALF + off, _LANES)]
                bx = geo[pl.ds(2 * _HALF + off, _LANES)]
                by = geo[pl.ds(3 * _HALF + off, _LANES)]
                e12t = t2v - t1v
                e13t = t3v - t1v
                e12p = d2p - d1p
                e13p = d3p - d1p
                cz = ax * by - ay * bx
                cxt = ay * e13t - e12t * by
                cyt = e12t * bx - ax * e13t
                cxp = ay * e13p - e12p * by
                cyp = e12p * bx - ax * e13p
                st = cxt * cxt + cyt * cyt + cz * cz
                sp = cxp * cxp + cyp * cyp + cz * cz
                mask = ((st > _THRESH2) & (t1v > 0.0)
                        & (t2v > 0.0) & (t3v > 0.0))
                ft = _rsqrt(jnp.maximum(st, 1e-24))
                fp = _rsqrt(jnp.maximum(sp, 1e-24))
                contrib = (jnp.abs(cxp * fp - cxt * ft)
                           + jnp.abs(cyp * fp - cyt * ft)
                           + jnp.abs(cz * fp - cz * ft))
                s_acc = s_acc + jnp.where(mask, contrib, 0.0)
                c_acc = c_acc + jnp.where(mask, 1.0, 0.0)
                return s_acc, c_acc

            zero = jnp.zeros((_LANES,), f32)
            s_acc, c_acc = lax.fori_loop(0, _HCHUNKS, body, (zero, zero))
            acc[pl.ds(0, _LANES)] = acc[pl.ds(0, _LANES)] + s_acc
            acc[pl.ds(_LANES, _LANES)] = (acc[pl.ds(_LANES, _LANES)]
                                          + c_acc)

        acc[pl.ds(0, _LANES)] = jnp.zeros((_LANES,), f32)
        acc[pl.ds(_LANES, _LANES)] = jnp.zeros((_LANES,), f32)

        for cp in idx_copies(t0, 0):
            cp.start()
        geo_copy(t0, 0).start()

        @pl.when(is_fast)
        def _():
            for cp in idx_copies(t1, 1):
                cp.start()
            geo_copy(t1, 1).start()

        for cp in idx_copies(t0, 0):
            cp.wait()
        for cp in gather_copies(0):
            cp.start()

        @pl.when(is_fast)
        def _():
            for cp in idx_copies(t1, 1):
                cp.wait()
            for cp in gather_copies(1):
                cp.start()

        for cp in gather_copies(0):
            cp.wait()
        geo_copy(t0, 0).wait()
        compute(0)

        @pl.when(is_fast)
        def _():
            for cp in idx_copies(t2, 2):
                cp.start()
            geo_copy(t2, 2).start()
            for cp in idx_copies(t2, 2):
                cp.wait()
            for cp in gather_copies(2):
                cp.start()
            for cp in gather_copies(1):
                cp.wait()
            geo_copy(t1, 1).wait()
            compute(1)
            for cp in gather_copies(2):
                cp.wait()
            geo_copy(t2, 2).wait()
            compute(2)

        pltpu.sync_copy(acc, out_h.at[wid])

    return sc_kernel


def _reduce_body(p_ref, o_ref):
    total = jnp.sum(p_ref[:, 0:_LANES])
    valid = jnp.sum(p_ref[:, _LANES:2 * _LANES])
    res = total / jnp.maximum(valid * 3.0, 1.0)
    o_ref[...] = jnp.reshape(res, (1, 1))


def kernel(pred, target):
    pred_f = pred.reshape(-1)
    targ_f = target.reshape(-1)
    consts = _host_constants()
    if consts is None:
        icat, geo = _pack_constants(_prep_constants())
    else:
        icat, geo = (jnp.asarray(c) for c in consts)
    parts = _make_sc_kernel()(pred_f, targ_f, icat, geo)
    out = pl.pallas_call(
        _reduce_body,
        out_shape=jax.ShapeDtypeStruct((1, 1), jnp.float32),
    )(parts)
    return out[0, 0]


# speedup vs baseline: 1.2872x; 1.2872x over previous
import functools

import jax
import jax.numpy as jnp
import numpy as np
from jax import lax
from jax.experimental import pallas as pl
from jax.experimental.pallas import tpu as pltpu
from jax.experimental.pallas import tpu_sc as plsc

_N = 16
_W = 384
_H = 384
_NUM_SAMPLES = 2000
_TOTAL = _N * _NUM_SAMPLES
_NS = 16
_NW = 32
_PAD = 32768
_HALF = 512
_NTASK = _PAD // _HALF
_LANES = 16
_HCHUNKS = _HALF // _LANES
_FAST_C = 1
_THRESH2 = float(np.float32(0.1) * np.float32(0.1))


def _prep_constants():
    key = jax.random.key(42)
    k1, k2, k3 = jax.random.split(key, 3)
    us, vs = [], []
    for k in (k1, k2, k3):
        ku, kv = jax.random.split(k)
        us.append(jax.random.randint(ku, (_N, _NUM_SAMPLES), 0, _W))
        vs.append(jax.random.randint(kv, (_N, _NUM_SAMPLES), 0, _H))
    boff = (jnp.arange(_N, dtype=jnp.int32) * (_W * _H))[:, None]
    idx = [(boff + u * _H + v).reshape(-1) for u, v in zip(us, vs)]
    uf = [(u.astype(jnp.float32) / _W).reshape(-1) for u in us]
    vf = [(v.astype(jnp.float32) / _H).reshape(-1) for v in vs]
    dx12 = uf[1] - uf[0]
    dy12 = vf[1] - vf[0]
    dx13 = uf[2] - uf[0]
    dy13 = vf[2] - vf[0]
    npad = _PAD - _TOTAL
    spread = (jnp.arange(npad, dtype=jnp.int32) * 16) % (_W * _H)

    def pad_idx(a, j):
        return jnp.concatenate([a, spread + j * 4096])

    def pad_zero(a):
        return jnp.concatenate([a, jnp.zeros((npad,), a.dtype)])

    return tuple(
        [pad_idx(idx[j], j) for j in range(3)]
        + [pad_zero(a) for a in (dx12, dy12, dx13, dy13)])


def _pack_constants(vals):
    i1, i2, i3, ax, ay, bx, by = vals
    icat = jnp.stack([v.reshape(_NTASK, _HALF) for v in (i1, i2, i3)],
                     axis=1).reshape(-1)
    geo = jnp.stack([v.reshape(_NTASK, _HALF) for v in (ax, ay, bx, by)],
                    axis=1).reshape(-1)
    return icat, geo


@functools.cache
def _host_constants():
    try:
        try:
            dev = jax.devices("cpu")[0]
        except RuntimeError:
            dev = None
        with jax.ensure_compile_time_eval():
            if dev is not None:
                with jax.default_device(dev):
                    vals = _pack_constants(_prep_constants())
            else:
                vals = _pack_constants(_prep_constants())
        return tuple(np.asarray(v) for v in jax.device_get(vals))
    except Exception:
        return None


def _rsqrt(s):
    i = lax.bitcast_convert_type(s, jnp.int32)
    i = jnp.int32(0x5F3759DF) - (i >> 1)
    y = lax.bitcast_convert_type(i, jnp.float32)
    for _ in range(3):
        y = y * (1.5 - 0.5 * s * y * y)
    return y


@functools.cache
def _make_sc_kernel():
    mesh = plsc.VectorSubcoreMesh(core_axis_name="c", subcore_axis_name="s")
    f32, i32 = jnp.float32, jnp.int32

    @functools.partial(
        pl.kernel,
        mesh=mesh,
        out_type=jax.ShapeDtypeStruct((_NW, 2 * _LANES), f32),
        scratch_types=[
            pltpu.VMEM((_HALF,), i32),
            pltpu.VMEM((_HALF,), i32),
            pltpu.VMEM((_HALF,), i32),
            pltpu.VMEM((_HALF,), i32),
            pltpu.VMEM((_HALF,), i32),
            pltpu.VMEM((_HALF,), i32),
            pltpu.VMEM((4 * _HALF,), f32),
            pltpu.VMEM((4 * _HALF,), f32),
            pltpu.VMEM((_HALF,), f32),
            pltpu.VMEM((_HALF,), f32),
            pltpu.VMEM((_HALF,), f32),
            pltpu.VMEM((_HALF,), f32),
            pltpu.VMEM((_HALF,), f32),
            pltpu.VMEM((_HALF,), f32),
            pltpu.VMEM((_HALF,), f32),
            pltpu.VMEM((_HALF,), f32),
            pltpu.VMEM((_HALF,), f32),
            pltpu.VMEM((_HALF,), f32),
            pltpu.VMEM((_HALF,), f32),
            pltpu.VMEM((_HALF,), f32),
            pltpu.VMEM((2 * _LANES,), f32),
            pltpu.SemaphoreType.DMA,
            pltpu.SemaphoreType.DMA,
            pltpu.SemaphoreType.DMA,
            pltpu.SemaphoreType.DMA,
            pltpu.SemaphoreType.DMA,
            pltpu.SemaphoreType.DMA,
            pltpu.SemaphoreType.DMA,
            pltpu.SemaphoreType.DMA,
            pltpu.SemaphoreType.DMA,
        ],
    )
    def sc_kernel(pred_hbm, targ_hbm, icat_h, geo_h, out_h,
                  i1a, i2a, i3a, i1b, i2b, i3b, geo_a, geo_b,
                  gp1a, gp2a, gp3a, gt1a, gt2a, gt3a,
                  gp1b, gp2b, gp3b, gt1b, gt2b, gt3b,
                  acc, sem_i0, sem_i1, sem_i2,
                  sem_geo0, sem_geo1, sem_geo2,
                  sem_g0, sem_g1, sem_g2):
        c = lax.axis_index("c")
        s = lax.axis_index("s")
        is_fast = c == _FAST_C
        wid = c * _NS + s
        t0 = jnp.where(is_fast, 3 * s, 3 * _NS + s)
        t1 = t0 + 1
        t2 = t0 + 2

        iset = ((i1a, i2a, i3a), (i1b, i2b, i3b))
        gset = ((gp1a, gp2a, gp3a, gt1a, gt2a, gt3a),
                (gp1b, gp2b, gp3b, gt1b, gt2b, gt3b))
        geos = (geo_a, geo_b)
        isems = (sem_i0, sem_i1, sem_i2)
        gsems = (sem_geo0, sem_geo1, sem_geo2)
        dsems = (sem_g0, sem_g1, sem_g2)

        def idx_copies(t, k):
            ib = 3 * _HALF * t
            return [pltpu.make_async_copy(
                        icat_h.at[pl.ds(ib + j * _HALF, _HALF)],
                        iset[k % 2][j], isems[k])
                    for j in range(3)]

        def geo_copy(t, k):
            return pltpu.make_async_copy(
                geo_h.at[pl.ds(4 * _HALF * t, 4 * _HALF)], geos[k % 2],
                gsems[k])

        def gather_copies(k):
            ivs = iset[k % 2]
            gvs = gset[k % 2]
            cps = [pltpu.make_async_copy(pred_hbm.at[ivs[j]], gvs[j],
                                         dsems[k]) for j in range(3)]
            cps += [pltpu.make_async_copy(targ_hbm.at[ivs[j]], gvs[j + 3],
                                          dsems[k]) for j in range(3)]
            return cps

        def compute(k):
            gp1, gp2, gp3, gt1, gt2, gt3 = gset[k % 2]
            geo = geos[k % 2]

            def body(i, carry):
                s_acc, c_acc = carry
                off = i * _LANES
                d1p = gp1[pl.ds(off, _LANES)]
                d2p = gp2[pl.ds(off, _LANES)]
                d3p = gp3[pl.ds(off, _LANES)]
                t1v = gt1[pl.ds(off, _LANES)]
                t2v = gt2[pl.ds(off, _LANES)]
                t3v = gt3[pl.ds(off, _LANES)]
                ax = geo[pl.ds(off, _LANES)]
                ay = geo[pl.ds(_HALF + off, _LANES)]
                bx = geo[pl.ds(2 * _HALF + off, _LANES)]
                by = geo[pl.ds(3 * _HALF + off, _LANES)]
                e12t = t2v - t1v
                e13t = t3v - t1v
                e12p = d2p - d1p
                e13p = d3p - d1p
                cz = ax * by - ay * bx
                cxt = ay * e13t - e12t * by
                cyt = e12t * bx - ax * e13t
                cxp = ay * e13p - e12p * by
                cyp = e12p * bx - ax * e13p
                st = cxt * cxt + cyt * cyt + cz * cz
                sp = cxp * cxp + cyp * cyp + cz * cz
                mask = ((st > _THRESH2) & (t1v > 0.0)
                        & (t2v > 0.0) & (t3v > 0.0))
                ft = _rsqrt(jnp.maximum(st, 1e-24))
                fp = _rsqrt(jnp.maximum(sp, 1e-24))
                contrib = (jnp.abs(cxp * fp - cxt * ft)
                           + jnp.abs(cyp * fp - cyt * ft)
                           + jnp.abs(cz * fp - cz * ft))
                s_acc = s_acc + jnp.where(mask, contrib, 0.0)
                c_acc = c_acc + jnp.where(mask, 1.0, 0.0)
                return s_acc, c_acc

            zero = jnp.zeros((_LANES,), f32)
            s_acc, c_acc = lax.fori_loop(0, _HCHUNKS, body, (zero, zero))
            acc[pl.ds(0, _LANES)] = acc[pl.ds(0, _LANES)] + s_acc
            acc[pl.ds(_LANES, _LANES)] = (acc[pl.ds(_LANES, _LANES)]
                                          + c_acc)

        acc[pl.ds(0, _LANES)] = jnp.zeros((_LANES,), f32)
        acc[pl.ds(_LANES, _LANES)] = jnp.zeros((_LANES,), f32)

        for cp in idx_copies(t0, 0):
            cp.start()
        geo_copy(t0, 0).start()

        @pl.when(is_fast)
        def _():
            for cp in idx_copies(t1, 1):
                cp.start()
            geo_copy(t1, 1).start()

        for cp in idx_copies(t0, 0):
            cp.wait()
        for cp in gather_copies(0):
            cp.start()

        @pl.when(is_fast)
        def _():
            for cp in idx_copies(t1, 1):
                cp.wait()
            for cp in gather_copies(1):
                cp.start()

        for cp in gather_copies(0):
            cp.wait()
        geo_copy(t0, 0).wait()
        compute(0)

        @pl.when(is_fast)
        def _():
            for cp in idx_copies(t2, 2):
                cp.start()
            geo_copy(t2, 2).start()
            for cp in idx_copies(t2, 2):
                cp.wait()
            for cp in gather_copies(2):
                cp.start()
            for cp in gather_copies(1):
                cp.wait()
            geo_copy(t1, 1).wait()
            compute(1)
            for cp in gather_copies(2):
                cp.wait()
            geo_copy(t2, 2).wait()
            compute(2)

        pltpu.sync_copy(acc, out_h.at[wid])

    return sc_kernel


def _reduce_body(p_ref, o_ref):
    total = jnp.sum(p_ref[:, 0:_LANES])
    valid = jnp.sum(p_ref[:, _LANES:2 * _LANES])
    res = total / jnp.maximum(valid * 3.0, 1.0)
    o_ref[...] = jnp.reshape(res, (1, 1))


def kernel(pred, target):
    pred_f = pred.reshape(-1)
    targ_f = target.reshape(-1)
    consts = _host_constants()
    if consts is None:
        icat, geo = _pack_constants(_prep_constants())
    else:
        icat, geo = (jnp.asarray(c) for c in consts)
    parts = _make_sc_kernel()(pred_f, targ_f, icat, geo)
    out = pl.pallas_call(
        _reduce_body,
        out_shape=jax.ShapeDtypeStruct((1, 1), jnp.float32),
    )(parts)
    return out[0, 0]


# speedup vs baseline: 1.3533x; 1.0513x over previous
import functools

import jax
import jax.numpy as jnp
import numpy as np
from jax import lax
from jax.experimental import pallas as pl
from jax.experimental.pallas import tpu as pltpu
from jax.experimental.pallas import tpu_sc as plsc

_N = 16
_W = 384
_H = 384
_NUM_SAMPLES = 2000
_TOTAL = _N * _NUM_SAMPLES
_NS = 16
_NW = 32
_PAD = 32768
_HALF = 512
_NTASK = _PAD // _HALF
_LANES = 16
_HCHUNKS = _HALF // _LANES
_FAST_C = 1
_THRESH2 = float(np.float32(0.1) * np.float32(0.1))


def _prep_constants():
    key = jax.random.key(42)
    k1, k2, k3 = jax.random.split(key, 3)
    us, vs = [], []
    for k in (k1, k2, k3):
        ku, kv = jax.random.split(k)
        us.append(jax.random.randint(ku, (_N, _NUM_SAMPLES), 0, _W))
        vs.append(jax.random.randint(kv, (_N, _NUM_SAMPLES), 0, _H))
    boff = (jnp.arange(_N, dtype=jnp.int32) * (_W * _H))[:, None]
    idx = [(boff + u * _H + v).reshape(-1) for u, v in zip(us, vs)]
    uf = [(u.astype(jnp.float32) / _W).reshape(-1) for u in us]
    vf = [(v.astype(jnp.float32) / _H).reshape(-1) for v in vs]
    dx12 = uf[1] - uf[0]
    dy12 = vf[1] - vf[0]
    dx13 = uf[2] - uf[0]
    dy13 = vf[2] - vf[0]
    npad = _PAD - _TOTAL
    spread = (jnp.arange(npad, dtype=jnp.int32) * 16) % (_W * _H)

    def pad_idx(a, j):
        return jnp.concatenate([a, spread + j * 4096])

    def pad_zero(a):
        return jnp.concatenate([a, jnp.zeros((npad,), a.dtype)])

    return tuple(
        [pad_idx(idx[j], j) for j in range(3)]
        + [pad_zero(a) for a in (dx12, dy12, dx13, dy13)])


def _pack_constants(vals):
    i1, i2, i3, ax, ay, bx, by = vals
    icat = jnp.stack([v.reshape(_NTASK, _HALF) for v in (i1, i2, i3)],
                     axis=1).reshape(-1)
    geo = jnp.stack([v.reshape(_NTASK, _HALF) for v in (ax, ay, bx, by)],
                    axis=1).reshape(-1)
    return icat, geo


@functools.cache
def _host_constants():
    try:
        try:
            dev = jax.devices("cpu")[0]
        except RuntimeError:
            dev = None
        with jax.ensure_compile_time_eval():
            if dev is not None:
                with jax.default_device(dev):
                    vals = _pack_constants(_prep_constants())
            else:
                vals = _pack_constants(_prep_constants())
        return tuple(np.asarray(v) for v in jax.device_get(vals))
    except Exception:
        return None


def _rsqrt(s):
    i = lax.bitcast_convert_type(s, jnp.int32)
    i = jnp.int32(0x5F3759DF) - (i >> 1)
    y = lax.bitcast_convert_type(i, jnp.float32)
    for _ in range(3):
        y = y * (1.5 - 0.5 * s * y * y)
    return y


@functools.cache
def _make_sc_kernel():
    mesh = plsc.VectorSubcoreMesh(core_axis_name="c", subcore_axis_name="s")
    f32, i32 = jnp.float32, jnp.int32

    @functools.partial(
        pl.kernel,
        mesh=mesh,
        out_type=jax.ShapeDtypeStruct((_NW, 2 * _LANES), f32),
        scratch_types=[
            pltpu.VMEM((_HALF,), i32),
            pltpu.VMEM((_HALF,), i32),
            pltpu.VMEM((_HALF,), i32),
            pltpu.VMEM((_HALF,), i32),
            pltpu.VMEM((_HALF,), i32),
            pltpu.VMEM((_HALF,), i32),
            pltpu.VMEM((4 * _HALF,), f32),
            pltpu.VMEM((4 * _HALF,), f32),
            pltpu.VMEM((_HALF,), f32),
            pltpu.VMEM((_HALF,), f32),
            pltpu.VMEM((_HALF,), f32),
            pltpu.VMEM((_HALF,), f32),
            pltpu.VMEM((_HALF,), f32),
            pltpu.VMEM((_HALF,), f32),
            pltpu.VMEM((_HALF,), f32),
            pltpu.VMEM((_HALF,), f32),
            pltpu.VMEM((_HALF,), f32),
            pltpu.VMEM((_HALF,), f32),
            pltpu.VMEM((_HALF,), f32),
            pltpu.VMEM((_HALF,), f32),
            pltpu.VMEM((2 * _LANES,), f32),
            pltpu.SemaphoreType.DMA,
            pltpu.SemaphoreType.DMA,
            pltpu.SemaphoreType.DMA,
            pltpu.SemaphoreType.DMA,
            pltpu.SemaphoreType.DMA,
            pltpu.SemaphoreType.DMA,
            pltpu.SemaphoreType.DMA,
            pltpu.SemaphoreType.DMA,
            pltpu.SemaphoreType.DMA,
        ],
    )
    def sc_kernel(pred_hbm, targ_hbm, icat_h, geo_h, out_h,
                  i1a, i2a, i3a, i1b, i2b, i3b, geo_a, geo_b,
                  gp1a, gp2a, gp3a, gt1a, gt2a, gt3a,
                  gp1b, gp2b, gp3b, gt1b, gt2b, gt3b,
                  acc, sem_i0, sem_i1, sem_i2,
                  sem_geo0, sem_geo1, sem_geo2,
                  sem_g0, sem_g1, sem_g2):
        c = lax.axis_index("c")
        s = lax.axis_index("s")
        wid = c * _NS + s
        t0 = 2 * wid
        t1 = t0 + 1

        iset = ((i1a, i2a, i3a), (i1b, i2b, i3b))
        gset = ((gp1a, gp2a, gp3a, gt1a, gt2a, gt3a),
                (gp1b, gp2b, gp3b, gt1b, gt2b, gt3b))
        geos = (geo_a, geo_b)
        isems = (sem_i0, sem_i1, sem_i2)
        gsems = (sem_geo0, sem_geo1, sem_geo2)
        dsems = (sem_g0, sem_g1, sem_g2)

        def idx_copies(t, k):
            ib = 3 * _HALF * t
            return [pltpu.make_async_copy(
                        icat_h.at[pl.ds(ib + j * _HALF, _HALF)],
                        iset[k % 2][j], isems[k])
                    for j in range(3)]

        def geo_copy(t, k):
            return pltpu.make_async_copy(
                geo_h.at[pl.ds(4 * _HALF * t, 4 * _HALF)], geos[k % 2],
                gsems[k])

        def gather_copies(k):
            ivs = iset[k % 2]
            gvs = gset[k % 2]
            cps = [pltpu.make_async_copy(pred_hbm.at[ivs[j]], gvs[j],
                                         dsems[k]) for j in range(3)]
            cps += [pltpu.make_async_copy(targ_hbm.at[ivs[j]], gvs[j + 3],
                                          dsems[k]) for j in range(3)]
            return cps

        def compute(k):
            gp1, gp2, gp3, gt1, gt2, gt3 = gset[k % 2]
            geo = geos[k % 2]

            def body(i, carry):
                s_acc, c_acc = carry
                off = i * _LANES
                d1p = gp1[pl.ds(off, _LANES)]
                d2p = gp2[pl.ds(off, _LANES)]
                d3p = gp3[pl.ds(off, _LANES)]
                t1v = gt1[pl.ds(off, _LANES)]
                t2v = gt2[pl.ds(off, _LANES)]
                t3v = gt3[pl.ds(off, _LANES)]
                ax = geo[pl.ds(off, _LANES)]
                ay = geo[pl.ds(_HALF + off, _LANES)]
                bx = geo[pl.ds(2 * _HALF + off, _LANES)]
                by = geo[pl.ds(3 * _HALF + off, _LANES)]
                e12t = t2v - t1v
                e13t = t3v - t1v
                e12p = d2p - d1p
                e13p = d3p - d1p
                cz = ax * by - ay * bx
                cxt = ay * e13t - e12t * by
                cyt = e12t * bx - ax * e13t
                cxp = ay * e13p - e12p * by
                cyp = e12p * bx - ax * e13p
                st = cxt * cxt + cyt * cyt + cz * cz
                sp = cxp * cxp + cyp * cyp + cz * cz
                mask = ((st > _THRESH2) & (t1v > 0.0)
                        & (t2v > 0.0) & (t3v > 0.0))
                ft = _rsqrt(jnp.maximum(st, 1e-24))
                fp = _rsqrt(jnp.maximum(sp, 1e-24))
                contrib = (jnp.abs(cxp * fp - cxt * ft)
                           + jnp.abs(cyp * fp - cyt * ft)
                           + jnp.abs(cz * fp - cz * ft))
                s_acc = s_acc + jnp.where(mask, contrib, 0.0)
                c_acc = c_acc + jnp.where(mask, 1.0, 0.0)
                return s_acc, c_acc

            zero = jnp.zeros((_LANES,), f32)
            s_acc, c_acc = lax.fori_loop(0, _HCHUNKS, body, (zero, zero))
            acc[pl.ds(0, _LANES)] = acc[pl.ds(0, _LANES)] + s_acc
            acc[pl.ds(_LANES, _LANES)] = (acc[pl.ds(_LANES, _LANES)]
                                          + c_acc)

        acc[pl.ds(0, _LANES)] = jnp.zeros((_LANES,), f32)
        acc[pl.ds(_LANES, _LANES)] = jnp.zeros((_LANES,), f32)

        for cp in idx_copies(t0, 0):
            cp.start()
        geo_copy(t0, 0).start()

        for cp in idx_copies(t1, 1):
            cp.start()
        geo_copy(t1, 1).start()

        for cp in idx_copies(t0, 0):
            cp.wait()
        for cp in gather_copies(0):
            cp.start()

        for cp in idx_copies(t1, 1):
            cp.wait()
        for cp in gather_copies(1):
            cp.start()

        for cp in gather_copies(0):
            cp.wait()
        geo_copy(t0, 0).wait()
        compute(0)

        for cp in gather_copies(1):
            cp.wait()
        geo_copy(t1, 1).wait()
        compute(1)

        pltpu.sync_copy(acc, out_h.at[wid])

    return sc_kernel


def _reduce_body(p_ref, o_ref):
    total = jnp.sum(p_ref[:, 0:_LANES])
    valid = jnp.sum(p_ref[:, _LANES:2 * _LANES])
    res = total / jnp.maximum(valid * 3.0, 1.0)
    o_ref[...] = jnp.reshape(res, (1, 1))


def kernel(pred, target):
    pred_f = pred.reshape(-1)
    targ_f = target.reshape(-1)
    consts = _host_constants()
    if consts is None:
        icat, geo = _pack_constants(_prep_constants())
    else:
        icat, geo = (jnp.asarray(c) for c in consts)
    parts = _make_sc_kernel()(pred_f, targ_f, icat, geo)
    out = pl.pallas_call(
        _reduce_body,
        out_shape=jax.ShapeDtypeStruct((1, 1), jnp.float32),
    )(parts)
    return out[0, 0]


# speedup vs baseline: 1.3552x; 1.0014x over previous
import functools

import jax
import jax.numpy as jnp
import numpy as np
from jax import lax
from jax.experimental import pallas as pl
from jax.experimental.pallas import tpu as pltpu
from jax.experimental.pallas import tpu_sc as plsc

_N = 16
_W = 384
_H = 384
_NUM_SAMPLES = 2000
_TOTAL = _N * _NUM_SAMPLES
_NS = 16
_NW = 32
_PAD = 32768
_HALF = 512
_NTASK = _PAD // _HALF
_LANES = 16
_HCHUNKS = _HALF // _LANES
_THRESH2 = float(np.float32(0.1) * np.float32(0.1))


def _prep_constants():
    key = jax.random.key(42)
    k1, k2, k3 = jax.random.split(key, 3)
    us, vs = [], []
    for k in (k1, k2, k3):
        ku, kv = jax.random.split(k)
        us.append(jax.random.randint(ku, (_N, _NUM_SAMPLES), 0, _W))
        vs.append(jax.random.randint(kv, (_N, _NUM_SAMPLES), 0, _H))
    boff = (jnp.arange(_N, dtype=jnp.int32) * (_W * _H))[:, None]
    idx = [(boff + u * _H + v).reshape(-1) for u, v in zip(us, vs)]
    uf = [(u.astype(jnp.float32) / _W).reshape(-1) for u in us]
    vf = [(v.astype(jnp.float32) / _H).reshape(-1) for v in vs]
    dx12 = uf[1] - uf[0]
    dy12 = vf[1] - vf[0]
    dx13 = uf[2] - uf[0]
    dy13 = vf[2] - vf[0]
    npad = _PAD - _TOTAL
    spread = (jnp.arange(npad, dtype=jnp.int32) * 16) % (_W * _H)

    def pad_idx(a, j):
        return jnp.concatenate([a, spread + j * 4096])

    def pad_zero(a):
        return jnp.concatenate([a, jnp.zeros((npad,), a.dtype)])

    return tuple(
        [pad_idx(idx[j], j) for j in range(3)]
        + [pad_zero(a) for a in (dx12, dy12, dx13, dy13)])


def _pack_constants(vals):
    i1, i2, i3, ax, ay, bx, by = vals
    icat = jnp.stack([v.reshape(_NTASK, _HALF) for v in (i1, i2, i3)],
                     axis=1).reshape(-1)
    geo = jnp.stack([v.reshape(_NTASK, _HALF) for v in (ax, ay, bx, by)],
                    axis=1).reshape(-1)
    return icat, geo


@functools.cache
def _host_constants():
    try:
        try:
            dev = jax.devices("cpu")[0]
        except RuntimeError:
            dev = None
        with jax.ensure_compile_time_eval():
            if dev is not None:
                with jax.default_device(dev):
                    vals = _pack_constants(_prep_constants())
            else:
                vals = _pack_constants(_prep_constants())
        return tuple(np.asarray(v) for v in jax.device_get(vals))
    except Exception:
        return None


def _rsqrt(s):
    i = lax.bitcast_convert_type(s, jnp.int32)
    i = jnp.int32(0x5F3759DF) - (i >> 1)
    y = lax.bitcast_convert_type(i, jnp.float32)
    for _ in range(3):
        y = y * (1.5 - 0.5 * s * y * y)
    return y


@functools.cache
def _make_sc_kernel():
    mesh = plsc.VectorSubcoreMesh(core_axis_name="c", subcore_axis_name="s")
    f32, i32 = jnp.float32, jnp.int32

    @functools.partial(
        pl.kernel,
        mesh=mesh,
        out_type=jax.ShapeDtypeStruct((_NW, 2 * _LANES), f32),
        scratch_types=[
            pltpu.VMEM((_HALF,), i32),
            pltpu.VMEM((_HALF,), i32),
            pltpu.VMEM((_HALF,), i32),
            pltpu.VMEM((_HALF,), i32),
            pltpu.VMEM((_HALF,), i32),
            pltpu.VMEM((_HALF,), i32),
            pltpu.VMEM((4 * _HALF,), f32),
            pltpu.VMEM((4 * _HALF,), f32),
            pltpu.VMEM((_HALF,), f32),
            pltpu.VMEM((_HALF,), f32),
            pltpu.VMEM((_HALF,), f32),
            pltpu.VMEM((_HALF,), f32),
            pltpu.VMEM((_HALF,), f32),
            pltpu.VMEM((_HALF,), f32),
            pltpu.VMEM((_HALF,), f32),
            pltpu.VMEM((_HALF,), f32),
            pltpu.VMEM((_HALF,), f32),
            pltpu.VMEM((_HALF,), f32),
            pltpu.VMEM((_HALF,), f32),
            pltpu.VMEM((_HALF,), f32),
            pltpu.VMEM((2 * _LANES,), f32),
            pltpu.SemaphoreType.DMA,
            pltpu.SemaphoreType.DMA,
            pltpu.SemaphoreType.DMA,
            pltpu.SemaphoreType.DMA,
            pltpu.SemaphoreType.DMA,
            pltpu.SemaphoreType.DMA,
        ],
    )
    def sc_kernel(pred_hbm, targ_hbm, icat_h, geo_h, out_h,
                  i1a, i2a, i3a, i1b, i2b, i3b, geo_a, geo_b,
                  gp1a, gp2a, gp3a, gt1a, gt2a, gt3a,
                  gp1b, gp2b, gp3b, gt1b, gt2b, gt3b,
                  acc, sem_i0, sem_i1,
                  sem_geo0, sem_geo1,
                  sem_g0, sem_g1):
        c = lax.axis_index("c")
        s = lax.axis_index("s")
        wid = c * _NS + s
        t0 = 2 * wid
        t1 = t0 + 1

        iset = ((i1a, i2a, i3a), (i1b, i2b, i3b))
        gset = ((gp1a, gp2a, gp3a, gt1a, gt2a, gt3a),
                (gp1b, gp2b, gp3b, gt1b, gt2b, gt3b))
        geos = (geo_a, geo_b)
        isems = (sem_i0, sem_i1)
        gsems = (sem_geo0, sem_geo1)
        dsems = (sem_g0, sem_g1)

        def idx_copies(t, k):
            ib = 3 * _HALF * t
            return [pltpu.make_async_copy(
                        icat_h.at[pl.ds(ib + j * _HALF, _HALF)],
                        iset[k % 2][j], isems[k])
                    for j in range(3)]

        def geo_copy(t, k):
            return pltpu.make_async_copy(
                geo_h.at[pl.ds(4 * _HALF * t, 4 * _HALF)], geos[k % 2],
                gsems[k])

        def gather_copies(k):
            ivs = iset[k % 2]
            gvs = gset[k % 2]
            cps = [pltpu.make_async_copy(pred_hbm.at[ivs[j]], gvs[j],
                                         dsems[k]) for j in range(3)]
            cps += [pltpu.make_async_copy(targ_hbm.at[ivs[j]], gvs[j + 3],
                                          dsems[k]) for j in range(3)]
            return cps

        def compute(k):
            gp1, gp2, gp3, gt1, gt2, gt3 = gset[k % 2]
            geo = geos[k % 2]

            def body(i, carry):
                s_acc, c_acc = carry
                off = i * _LANES
                d1p = gp1[pl.ds(off, _LANES)]
                d2p = gp2[pl.ds(off, _LANES)]
                d3p = gp3[pl.ds(off, _LANES)]
                t1v = gt1[pl.ds(off, _LANES)]
                t2v = gt2[pl.ds(off, _LANES)]
                t3v = gt3[pl.ds(off, _LANES)]
                ax = geo[pl.ds(off, _LANES)]
                ay = geo[pl.ds(_HALF + off, _LANES)]
                bx = geo[pl.ds(2 * _HALF + off, _LANES)]
                by = geo[pl.ds(3 * _HALF + off, _LANES)]
                e12t = t2v - t1v
                e13t = t3v - t1v
                e12p = d2p - d1p
                e13p = d3p - d1p
                cz = ax * by - ay * bx
                cxt = ay * e13t - e12t * by
                cyt = e12t * bx - ax * e13t
                cxp = ay * e13p - e12p * by
                cyp = e12p * bx - ax * e13p
                st = cxt * cxt + cyt * cyt + cz * cz
                sp = cxp * cxp + cyp * cyp + cz * cz
                mask = ((st > _THRESH2) & (t1v > 0.0)
                        & (t2v > 0.0) & (t3v > 0.0))
                ft = _rsqrt(jnp.maximum(st, 1e-24))
                fp = _rsqrt(jnp.maximum(sp, 1e-24))
                contrib = (jnp.abs(cxp * fp - cxt * ft)
                           + jnp.abs(cyp * fp - cyt * ft)
                           + jnp.abs(cz * fp - cz * ft))
                s_acc = s_acc + jnp.where(mask, contrib, 0.0)
                c_acc = c_acc + jnp.where(mask, 1.0, 0.0)
                return s_acc, c_acc

            zero = jnp.zeros((_LANES,), f32)
            s_acc, c_acc = lax.fori_loop(0, _HCHUNKS, body, (zero, zero))
            acc[pl.ds(0, _LANES)] = acc[pl.ds(0, _LANES)] + s_acc
            acc[pl.ds(_LANES, _LANES)] = (acc[pl.ds(_LANES, _LANES)]
                                          + c_acc)

        acc[pl.ds(0, _LANES)] = jnp.zeros((_LANES,), f32)
        acc[pl.ds(_LANES, _LANES)] = jnp.zeros((_LANES,), f32)

        for cp in idx_copies(t0, 0):
            cp.start()
        geo_copy(t0, 0).start()

        for cp in idx_copies(t1, 1):
            cp.start()
        geo_copy(t1, 1).start()

        for cp in idx_copies(t0, 0):
            cp.wait()
        for cp in gather_copies(0):
            cp.start()

        for cp in idx_copies(t1, 1):
            cp.wait()
        for cp in gather_copies(1):
            cp.start()

        for cp in gather_copies(0):
            cp.wait()
        geo_copy(t0, 0).wait()
        compute(0)

        for cp in gather_copies(1):
            cp.wait()
        geo_copy(t1, 1).wait()
        compute(1)

        pltpu.sync_copy(acc, out_h.at[wid])

    return sc_kernel


def _reduce_body(p_ref, o_ref):
    total = jnp.sum(p_ref[:, 0:_LANES])
    valid = jnp.sum(p_ref[:, _LANES:2 * _LANES])
    res = total / jnp.maximum(valid * 3.0, 1.0)
    o_ref[...] = jnp.reshape(res, (1, 1))


def kernel(pred, target):
    pred_f = pred.reshape(-1)
    targ_f = target.reshape(-1)
    consts = _host_constants()
    if consts is None:
        icat, geo = _pack_constants(_prep_constants())
    else:
        icat, geo = (jnp.asarray(c) for c in consts)
    parts = _make_sc_kernel()(pred_f, targ_f, icat, geo)
    out = pl.pallas_call(
        _reduce_body,
        out_shape=jax.ShapeDtypeStruct((1, 1), jnp.float32),
    )(parts)
    return out[0, 0]
